# Initial kernel scaffold; baseline (speedup 1.0000x reference)
import jax
import jax.numpy as jnp
from jax.experimental import pallas as pl


def kernel(coords, block_pair_dispatch_indices, pose_stack_block_coord_offset, pose_stack_block_type, pose_stack_min_bond_separation, bt_tile_n_donH, bt_tile_n_acc, bt_tile_donH_inds, bt_tile_acc_inds, bt_tile_donor_type, bt_tile_acceptor_type, bt_tile_acceptor_hybridization, bt_atom_is_hydrogen, bt_path_distance, pair_params, pair_polynomials, global_params):
    raise NotImplementedError("write your pallas kernel here")


# speedup vs baseline: 186.4449x; 186.4449x over previous
import functools

import jax
import jax.numpy as jnp
from jax import lax
from jax.experimental import pallas as pl
from jax.experimental.pallas import tpu as pltpu
from jax.experimental.pallas import tpu_sc as plsc

_NC = 2
_NS = 16
_NW = _NC * _NS
_LN = 16
_NCH = 128
_MAXD = 4
_MAXA = 4
_DEG = 11
_CPAD = 16


def _rsqrt(x):
    i = plsc.bitcast(x, jnp.int32)
    i = 0x5F3759DF - lax.shift_right_logical(i, 1)
    y = plsc.bitcast(i, jnp.float32)
    for _ in range(4):
        y = y * (1.5 - 0.5 * x * y * y)
    return y


def _body(n_pair, n_pose, n_blk, n_tile, n_at, wp, nch_w,
          dp_h, dbi_h, dbj_h, bty_h, off_h, donh_h, acc_h, dty_h, aty_h,
          hyb_h, ish_h, ndon_h, nacc_h, pathd_h, coef_h, pp0_h, pp1_h,
          sep_h, g_h, crd_h,
          out_h,
          dp_v, dbi_v, dbj_v, bty_v, off_v, donh_v, acc_v, dty_v, aty_v,
          hyb_v, ish_v, ndon_v, nacc_v, pathd_v, coef_v, pp0_v, pp1_v,
          g_v, idxc, idxs, btib, btjb, cbuf, sbuf, tot_v, orow, sem):
    wid = lax.axis_index("s") * _NC + lax.axis_index("c")
    wbase = wid * wp

    pltpu.sync_copy(dp_h.at[pl.ds(wbase, wp)], dp_v)
    pltpu.sync_copy(dbi_h.at[pl.ds(wbase, wp)], dbi_v)
    pltpu.sync_copy(dbj_h.at[pl.ds(wbase, wp)], dbj_v)
    pltpu.sync_copy(bty_h, bty_v)
    pltpu.sync_copy(off_h, off_v)
    pltpu.sync_copy(donh_h, donh_v)
    pltpu.sync_copy(acc_h, acc_v)
    pltpu.sync_copy(dty_h, dty_v)
    pltpu.sync_copy(aty_h, aty_v)
    pltpu.sync_copy(hyb_h, hyb_v)
    pltpu.sync_copy(ish_h, ish_v)
    pltpu.sync_copy(ndon_h, ndon_v)
    pltpu.sync_copy(nacc_h, nacc_v)
    pltpu.sync_copy(pathd_h, pathd_v)
    pltpu.sync_copy(coef_h, coef_v)
    pltpu.sync_copy(pp0_h, pp0_v)
    pltpu.sync_copy(pp1_h, pp1_v)
    pltpu.sync_copy(g_h, g_v)

    iota = lax.iota(jnp.int32, _LN)
    zero16 = jnp.zeros((_LN,), jnp.float32)

    def zero_body(i, _):
        tot_v[pl.ds(i * _LN, _LN)] = zero16
        return 0
    lax.fori_loop(0, n_pose, zero_body, 0)

    nvec = _NCH // _LN

    def chunk_body(c, _):
        cb = c * _NCH

        def a_body(v, _):
            b = cb + v * _LN
            vb = v * _LN
            pv = dp_v[pl.ds(b, _LN)]
            biv = dbi_v[pl.ds(b, _LN)]
            bjv = dbj_v[pl.ds(b, _LN)]
            pb = pv * n_blk + biv
            pbj = pv * n_blk + bjv
            bti = plsc.load_gather(bty_v, [pb])
            btj = plsc.load_gather(bty_v, [pbj])
            offi = plsc.load_gather(off_v, [pb])
            offj = plsc.load_gather(off_v, [pbj])
            ci = pv * n_at + offi
            cj = pv * n_at + offj
            btib[pl.ds(vb, _LN)] = bti
            btjb[pl.ds(vb, _LN)] = btj
            for k in range(_MAXD):
                hk = plsc.load_gather(donh_v, [bti * n_tile + k])
                idxc[pl.ds(k * _NCH + vb, _LN)] = ci + hk
            for a in range(_MAXA):
                ak = plsc.load_gather(acc_v, [btj * n_tile + a])
                idxc[pl.ds((_MAXD + a) * _NCH + vb, _LN)] = cj + ak
            idxs[pl.ds(vb, _LN)] = lax.shift_right_logical(
                pb * n_blk + bjv, 4)
            return 0
        lax.fori_loop(0, nvec, a_body, 0)

        cps = []
        for k in range(_MAXD + _MAXA):
            cps.append(pltpu.async_copy(
                crd_h.at[idxc.at[pl.ds(k * _NCH, _NCH)]], cbuf.at[k], sem))
        cps.append(pltpu.async_copy(sep_h.at[idxs], sbuf, sem))
        for cp in cps:
            cp.wait()

        def b_body(v, _):
            b = cb + v * _LN
            vb = v * _LN
            j16 = vb + iota
            gidx = wbase + b + iota
            valid = gidx < n_pair
            pv = dp_v[pl.ds(b, _LN)]
            biv = dbi_v[pl.ds(b, _LN)]
            bjv = dbj_v[pl.ds(b, _LN)]
            bti = btib[pl.ds(vb, _LN)]
            btj = btjb[pl.ds(vb, _LN)]
            nH = plsc.load_gather(ndon_v, [bti])
            nA = plsc.load_gather(nacc_v, [btj])
            flat = (pv * n_blk + biv) * n_blk + bjv
            sep_inter = plsc.load_gather(sbuf, [j16, flat & 15])
            intra = biv == bjv

            Hx, Hy, Hz, hidx, ishk, dtk = [], [], [], [], [], []
            for k in range(_MAXD):
                kc = jnp.full((_LN,), k, jnp.int32)
                hk = plsc.load_gather(donh_v, [bti * n_tile + k])
                hidx.append(hk)
                ishk.append(plsc.load_gather(ish_v, [bti * n_tile + hk]) > 0)
                dtk.append(plsc.load_gather(dty_v, [bti * n_tile + k]))
                Hx.append(plsc.load_gather(
                    cbuf, [kc, j16, jnp.full((_LN,), 0, jnp.int32)]))
                Hy.append(plsc.load_gather(
                    cbuf, [kc, j16, jnp.full((_LN,), 1, jnp.int32)]))
                Hz.append(plsc.load_gather(
                    cbuf, [kc, j16, jnp.full((_LN,), 2, jnp.int32)]))
            Ax, Ay, Az, aidx, atk, fade = [], [], [], [], [], []
            for a in range(_MAXA):
                ac = jnp.full((_LN,), _MAXD + a, jnp.int32)
                ak = plsc.load_gather(acc_v, [btj * n_tile + a])
                aidx.append(ak)
                atk.append(plsc.load_gather(aty_v, [btj * n_tile + a]))
                hy = plsc.load_gather(hyb_v, [btj * n_tile + a])
                fade.append(1.0 + 0.1 * hy.astype(jnp.float32))
                Ax.append(plsc.load_gather(
                    cbuf, [ac, j16, jnp.full((_LN,), 0, jnp.int32)]))
                Ay.append(plsc.load_gather(
                    cbuf, [ac, j16, jnp.full((_LN,), 1, jnp.int32)]))
                Az.append(plsc.load_gather(
                    cbuf, [ac, j16, jnp.full((_LN,), 2, jnp.int32)]))

            acc = zero16
            for k in range(_MAXD):
                hv = nH > k
                for a in range(_MAXA):
                    av = nA > a
                    dx = Ax[a] - Hx[k]
                    dy = Ay[a] - Hy[k]
                    dz = Az[a] - Hz[k]
                    s = dx * dx + dy * dy + dz * dz + 1e-8
                    r = _rsqrt(s)
                    d = s * r
                    pi = dtk[k] * 6 + atk[a]
                    x = d * (1.0 / 3.0)
                    val = plsc.load_gather(coef_v, [pi * _CPAD])
                    for q in range(1, _DEG):
                        val = val * x + plsc.load_gather(
                            coef_v, [pi * _CPAD + q])
                    p0 = plsc.load_gather(pp0_v, [pi])
                    p1 = plsc.load_gather(pp1_v, [pi])
                    lo = jnp.minimum(p0, p1)
                    hi = 3.0 * (1.0 + jnp.maximum(p0, p1))
                    si = plsc.load_gather(
                        pathd_v, [(bti * n_tile + hidx[k]) * n_tile + aidx[a]])
                    sep = jnp.where(intra, si, sep_inter)
                    m = (hv & av & (d > lo) & (d < hi) & (sep >= 4)
                         & ishk[k] & valid)
                    acc = acc + jnp.where(m, val * fade[a], 0.0)
            plsc.addupdate_scatter(tot_v, [pv * _LN + iota], acc)
            return 0
        lax.fori_loop(0, nvec, b_body, 0)
        return 0
    lax.fori_loop(0, nch_w, chunk_body, 0)

    gv = g_v[pl.ds(0, _LN)]
    for grp in range(n_pose // _LN):
        rows = grp * _LN + iota
        accg = zero16
        for j in range(_LN):
            accg = accg + plsc.load_gather(tot_v, [rows * _LN + j])
        orow[pl.ds(grp * _LN, _LN)] = accg * gv
    pltpu.sync_copy(orow, out_h.at[wid])


def kernel(coords, block_pair_dispatch_indices, pose_stack_block_coord_offset,
           pose_stack_block_type, pose_stack_min_bond_separation,
           bt_tile_n_donH, bt_tile_n_acc, bt_tile_donH_inds, bt_tile_acc_inds,
           bt_tile_donor_type, bt_tile_acceptor_type,
           bt_tile_acceptor_hybridization, bt_atom_is_hydrogen,
           bt_path_distance, pair_params, pair_polynomials, global_params):
    n_pose, n_at, _ = coords.shape
    n_blk = pose_stack_block_type.shape[1]
    n_bt, _, n_tile = bt_tile_donH_inds.shape
    n_pair = block_pair_dispatch_indices.shape[0]
    ndt, nat, deg = pair_polynomials.shape

    chunk_total = _NW * _NCH
    nch_w = -(-n_pair // chunk_total)
    wp = nch_w * _NCH
    tot = _NW * wp

    disp = block_pair_dispatch_indices.astype(jnp.int32)
    disp = jnp.pad(disp, ((0, tot - n_pair), (0, 0)))
    dp = disp[:, 0]
    dbi = disp[:, 1]
    dbj = disp[:, 2]

    crd = jnp.pad(coords.reshape(n_pose * n_at, 3), ((0, 0), (0, 13)))
    bty = pose_stack_block_type.reshape(-1).astype(jnp.int32)
    off = pose_stack_block_coord_offset.reshape(-1).astype(jnp.int32)
    sep = pose_stack_min_bond_separation.reshape(-1, 16).astype(jnp.int32)
    donh = bt_tile_donH_inds.reshape(-1).astype(jnp.int32)
    accs = bt_tile_acc_inds.reshape(-1).astype(jnp.int32)
    dty = bt_tile_donor_type.reshape(-1).astype(jnp.int32)
    aty = bt_tile_acceptor_type.reshape(-1).astype(jnp.int32)
    hyb = bt_tile_acceptor_hybridization.reshape(-1).astype(jnp.int32)
    ish = bt_atom_is_hydrogen.reshape(-1).astype(jnp.int32)
    npadbt = (-n_bt) % _LN
    ndon = jnp.pad(bt_tile_n_donH.reshape(-1), (0, npadbt)).astype(jnp.int32)
    nacc = jnp.pad(bt_tile_n_acc.reshape(-1), (0, npadbt)).astype(jnp.int32)
    pathd = bt_path_distance.reshape(-1).astype(jnp.int32)
    coef = jnp.pad(pair_polynomials.reshape(ndt * nat, deg),
                   ((0, 0), (0, _CPAD - deg))).reshape(-1)
    pp0 = jnp.pad(pair_params[..., 0].reshape(-1), (0, (-ndt * nat) % _LN))
    pp1 = jnp.pad(pair_params[..., 1].reshape(-1), (0, (-ndt * nat) % _LN))
    g16 = jnp.broadcast_to(global_params[0], (_LN,))

    mesh = plsc.VectorSubcoreMesh(core_axis_name="c", subcore_axis_name="s")
    f32 = jnp.float32
    i32 = jnp.int32
    run = pl.kernel(
        functools.partial(_body, n_pair, n_pose, n_blk, n_tile, n_at, wp,
                          nch_w),
        out_type=jax.ShapeDtypeStruct((_NW, n_pose), f32),
        mesh=mesh,
        compiler_params=pltpu.CompilerParams(
            needs_layout_passes=False, use_tc_tiling_on_sc=False),
        scratch_types=[
            pltpu.VMEM((wp,), i32),
            pltpu.VMEM((wp,), i32),
            pltpu.VMEM((wp,), i32),
            pltpu.VMEM((n_pose * n_blk,), i32),
            pltpu.VMEM((n_pose * n_blk,), i32),
            pltpu.VMEM((n_bt * n_tile,), i32),
            pltpu.VMEM((n_bt * n_tile,), i32),
            pltpu.VMEM((n_bt * n_tile,), i32),
            pltpu.VMEM((n_bt * n_tile,), i32),
            pltpu.VMEM((n_bt * n_tile,), i32),
            pltpu.VMEM((n_bt * n_tile,), i32),
            pltpu.VMEM((n_bt + npadbt,), i32),
            pltpu.VMEM((n_bt + npadbt,), i32),
            pltpu.VMEM((n_bt * n_tile * n_tile,), i32),
            pltpu.VMEM((ndt * nat * _CPAD,), f32),
            pltpu.VMEM((ndt * nat + (-ndt * nat) % _LN,), f32),
            pltpu.VMEM((ndt * nat + (-ndt * nat) % _LN,), f32),
            pltpu.VMEM((_LN,), f32),
            pltpu.VMEM(((_MAXD + _MAXA) * _NCH,), i32),
            pltpu.VMEM((_NCH,), i32),
            pltpu.VMEM((_NCH,), i32),
            pltpu.VMEM((_NCH,), i32),
            pltpu.VMEM((_MAXD + _MAXA, _NCH, 16), f32),
            pltpu.VMEM((_NCH, 16), i32),
            pltpu.VMEM((n_pose * _LN,), f32),
            pltpu.VMEM((n_pose,), f32),
            pltpu.SemaphoreType.DMA,
        ],
    )
    partial = run(dp, dbi, dbj, bty, off, donh, accs, dty, aty, hyb, ish,
                  ndon, nacc, pathd, coef, pp0, pp1, sep, g16, crd)
    return jnp.sum(partial, axis=0)


# speedup vs baseline: 211.3506x; 1.1336x over previous
import functools

import jax
import jax.numpy as jnp
from jax import lax
from jax.experimental import pallas as pl
from jax.experimental.pallas import tpu as pltpu
from jax.experimental.pallas import tpu_sc as plsc

_NC = 2
_NS = 16
_NW = _NC * _NS
_LN = 16
_NCH = 128
_MAXD = 4
_MAXA = 4
_DEG = 11
_CPAD = 16


def _rsqrt(x):
    i = plsc.bitcast(x, jnp.int32)
    i = 0x5F3759DF - lax.shift_right_logical(i, 1)
    y = plsc.bitcast(i, jnp.float32)
    for _ in range(3):
        y = y * (1.5 - 0.5 * x * y * y)
    return y


def _body(n_pair, n_pose, n_blk, n_tile, n_at, wp, nch_w,
          dp_h, dbi_h, dbj_h, bty_h, off_h, donh_h, acc_h, dty_h, aty_h,
          hyb_h, ish_h, ndon_h, nacc_h, pathd_h, coef_h, pp0_h, pp1_h,
          sep_h, g_h, crd_h,
          out_h,
          dp_v, dbi_v, dbj_v, bty_v, off_v, donh_v, acc_v, dty_v, aty_v,
          hyb_v, ish_v, ndon_v, nacc_v, pathd_v, coef_v, pp0_v, pp1_v,
          g_v, lo_v, hi_v, fade_v,
          idxc0, idxc1, idxs0, idxs1, btib0, btib1, btjb0, btjb1,
          cbuf0, cbuf1, sbuf0, sbuf1, tot_v, orow, sem0, sem1):
    wid = lax.axis_index("s") * _NC + lax.axis_index("c")
    wbase = wid * wp

    pltpu.sync_copy(dp_h.at[pl.ds(wbase, wp)], dp_v)
    pltpu.sync_copy(dbi_h.at[pl.ds(wbase, wp)], dbi_v)
    pltpu.sync_copy(dbj_h.at[pl.ds(wbase, wp)], dbj_v)
    pltpu.sync_copy(bty_h, bty_v)
    pltpu.sync_copy(off_h, off_v)
    pltpu.sync_copy(donh_h, donh_v)
    pltpu.sync_copy(acc_h, acc_v)
    pltpu.sync_copy(dty_h, dty_v)
    pltpu.sync_copy(aty_h, aty_v)
    pltpu.sync_copy(hyb_h, hyb_v)
    pltpu.sync_copy(ish_h, ish_v)
    pltpu.sync_copy(ndon_h, ndon_v)
    pltpu.sync_copy(nacc_h, nacc_v)
    pltpu.sync_copy(pathd_h, pathd_v)
    pltpu.sync_copy(coef_h, coef_v)
    pltpu.sync_copy(pp0_h, pp0_v)
    pltpu.sync_copy(pp1_h, pp1_v)
    pltpu.sync_copy(g_h, g_v)

    iota = lax.iota(jnp.int32, _LN)
    zero16 = jnp.zeros((_LN,), jnp.float32)

    npp = lo_v.shape[0] // _LN
    for i in range(npp):
        p0 = pp0_v[pl.ds(i * _LN, _LN)]
        p1 = pp1_v[pl.ds(i * _LN, _LN)]
        lo_v[pl.ds(i * _LN, _LN)] = jnp.minimum(p0, p1)
        hi_v[pl.ds(i * _LN, _LN)] = 3.0 * (1.0 + jnp.maximum(p0, p1))

    def fade_body(i, _):
        hy = hyb_v[pl.ds(i * _LN, _LN)]
        fade_v[pl.ds(i * _LN, _LN)] = 1.0 + 0.1 * hy.astype(jnp.float32)
        return 0
    lax.fori_loop(0, fade_v.shape[0] // _LN, fade_body, 0)

    def zero_body(i, _):
        tot_v[pl.ds(i * _LN, _LN)] = zero16
        return 0
    lax.fori_loop(0, n_pose, zero_body, 0)

    nvec = _NCH // _LN

    def a_phase(cc, idxc, idxs, btib, btjb):
        cb = cc * _NCH

        def a_body(v, _):
            b = cb + v * _LN
            vb = v * _LN
            pv = dp_v[pl.ds(b, _LN)]
            biv = dbi_v[pl.ds(b, _LN)]
            bjv = dbj_v[pl.ds(b, _LN)]
            pb = pv * n_blk + biv
            pbj = pv * n_blk + bjv
            bti = plsc.load_gather(bty_v, [pb])
            btj = plsc.load_gather(bty_v, [pbj])
            offi = plsc.load_gather(off_v, [pb])
            offj = plsc.load_gather(off_v, [pbj])
            ci = pv * n_at + offi
            cj = pv * n_at + offj
            btib[pl.ds(vb, _LN)] = bti
            btjb[pl.ds(vb, _LN)] = btj
            for k in range(_MAXD):
                hk = plsc.load_gather(donh_v, [bti * n_tile + k])
                idxc[pl.ds(k * _NCH + vb, _LN)] = ci + hk
            for a in range(_MAXA):
                ak = plsc.load_gather(acc_v, [btj * n_tile + a])
                idxc[pl.ds((_MAXD + a) * _NCH + vb, _LN)] = cj + ak
            idxs[pl.ds(vb, _LN)] = lax.shift_right_logical(
                pb * n_blk + bjv, 4)
            return 0
        lax.fori_loop(0, nvec, a_body, 0)

    def fire(idxc, idxs, cbuf, sbuf, sem):
        for k in range(_MAXD + _MAXA):
            pltpu.async_copy(
                crd_h.at[idxc.at[pl.ds(k * _NCH, _NCH)]], cbuf.at[k], sem)
        pltpu.async_copy(sep_h.at[idxs], sbuf, sem)

    def drain(idxc, idxs, cbuf, sbuf, sem):
        for k in range(_MAXD + _MAXA):
            pltpu.make_async_copy(
                crd_h.at[idxc.at[pl.ds(k * _NCH, _NCH)]],
                cbuf.at[k], sem).wait()
        pltpu.make_async_copy(sep_h.at[idxs], sbuf, sem).wait()

    def b_phase(cc, btib, btjb, cbuf, sbuf):
        cb = cc * _NCH

        def b_body(v, _):
            b = cb + v * _LN
            vb = v * _LN
            j16 = vb + iota
            gidx = wbase + b + iota
            valid = gidx < n_pair
            pv = dp_v[pl.ds(b, _LN)]
            biv = dbi_v[pl.ds(b, _LN)]
            bjv = dbj_v[pl.ds(b, _LN)]
            bti = btib[pl.ds(vb, _LN)]
            btj = btjb[pl.ds(vb, _LN)]
            nH = plsc.load_gather(ndon_v, [bti])
            nA = plsc.load_gather(nacc_v, [btj])
            flat = (pv * n_blk + biv) * n_blk + bjv
            sep_inter = plsc.load_gather(sbuf, [j16, flat & 15])
            intra = biv == bjv

            Hx, Hy, Hz, hidx, ishk, dtk = [], [], [], [], [], []
            for k in range(_MAXD):
                kc = jnp.full((_LN,), k, jnp.int32)
                hk = plsc.load_gather(donh_v, [bti * n_tile + k])
                hidx.append(hk)
                ishk.append(plsc.load_gather(ish_v, [bti * n_tile + hk]) > 0)
                dtk.append(plsc.load_gather(dty_v, [bti * n_tile + k]))
                Hx.append(plsc.load_gather(
                    cbuf, [kc, j16, jnp.full((_LN,), 0, jnp.int32)]))
                Hy.append(plsc.load_gather(
                    cbuf, [kc, j16, jnp.full((_LN,), 1, jnp.int32)]))
                Hz.append(plsc.load_gather(
                    cbuf, [kc, j16, jnp.full((_LN,), 2, jnp.int32)]))
            Ax, Ay, Az, aidx, atk, fade = [], [], [], [], [], []
            for a in range(_MAXA):
                ac = jnp.full((_LN,), _MAXD + a, jnp.int32)
                ak = plsc.load_gather(acc_v, [btj * n_tile + a])
                aidx.append(ak)
                atk.append(plsc.load_gather(aty_v, [btj * n_tile + a]))
                fade.append(plsc.load_gather(fade_v, [btj * n_tile + a]))
                Ax.append(plsc.load_gather(
                    cbuf, [ac, j16, jnp.full((_LN,), 0, jnp.int32)]))
                Ay.append(plsc.load_gather(
                    cbuf, [ac, j16, jnp.full((_LN,), 1, jnp.int32)]))
                Az.append(plsc.load_gather(
                    cbuf, [ac, j16, jnp.full((_LN,), 2, jnp.int32)]))

            acc = zero16
            for k in range(_MAXD):
                hv = nH > k
                for a in range(_MAXA):
                    av = nA > a
                    dx = Ax[a] - Hx[k]
                    dy = Ay[a] - Hy[k]
                    dz = Az[a] - Hz[k]
                    s = dx * dx + dy * dy + dz * dz + 1e-8
                    r = _rsqrt(s)
                    d = s * r
                    pi = dtk[k] * 6 + atk[a]
                    x = d * (1.0 / 3.0)
                    val = plsc.load_gather(coef_v, [pi * _CPAD])
                    for q in range(1, _DEG):
                        val = val * x + plsc.load_gather(
                            coef_v, [pi * _CPAD + q])
                    lo = plsc.load_gather(lo_v, [pi])
                    hi = plsc.load_gather(hi_v, [pi])
                    si = plsc.load_gather(
                        pathd_v, [(bti * n_tile + hidx[k]) * n_tile + aidx[a]])
                    sep = jnp.where(intra, si, sep_inter)
                    m = (hv & av & (d > lo) & (d < hi) & (sep >= 4)
                         & ishk[k] & valid)
                    acc = acc + jnp.where(m, val * fade[a], 0.0)
            plsc.addupdate_scatter(tot_v, [pv * _LN + iota], acc)
            return 0
        lax.fori_loop(0, nvec, b_body, 0)

    a_phase(0, idxc0, idxs0, btib0, btjb0)
    fire(idxc0, idxs0, cbuf0, sbuf0, sem0)

    def pair_body(g, _):
        c0 = 2 * g
        c1 = c0 + 1
        a_phase(c1, idxc1, idxs1, btib1, btjb1)
        fire(idxc1, idxs1, cbuf1, sbuf1, sem1)
        drain(idxc0, idxs0, cbuf0, sbuf0, sem0)
        b_phase(c0, btib0, btjb0, cbuf0, sbuf0)

        @pl.when(c0 + 2 < nch_w)
        def _():
            a_phase(c0 + 2, idxc0, idxs0, btib0, btjb0)
            fire(idxc0, idxs0, cbuf0, sbuf0, sem0)
        drain(idxc1, idxs1, cbuf1, sbuf1, sem1)
        b_phase(c1, btib1, btjb1, cbuf1, sbuf1)
        return 0
    lax.fori_loop(0, nch_w // 2, pair_body, 0)

    gv = g_v[pl.ds(0, _LN)]
    for grp in range(n_pose // _LN):
        rows = grp * _LN + iota
        accg = zero16
        for j in range(_LN):
            accg = accg + plsc.load_gather(tot_v, [rows * _LN + j])
        orow[pl.ds(grp * _LN, _LN)] = accg * gv
    pltpu.sync_copy(orow, out_h.at[wid])


def kernel(coords, block_pair_dispatch_indices, pose_stack_block_coord_offset,
           pose_stack_block_type, pose_stack_min_bond_separation,
           bt_tile_n_donH, bt_tile_n_acc, bt_tile_donH_inds, bt_tile_acc_inds,
           bt_tile_donor_type, bt_tile_acceptor_type,
           bt_tile_acceptor_hybridization, bt_atom_is_hydrogen,
           bt_path_distance, pair_params, pair_polynomials, global_params):
    n_pose, n_at, _ = coords.shape
    n_blk = pose_stack_block_type.shape[1]
    n_bt, _, n_tile = bt_tile_donH_inds.shape
    n_pair = block_pair_dispatch_indices.shape[0]
    ndt, nat, deg = pair_polynomials.shape

    chunk_total = _NW * _NCH
    nch_w = -(-n_pair // chunk_total)
    nch_w += nch_w % 2
    wp = nch_w * _NCH
    tot = _NW * wp

    disp = block_pair_dispatch_indices.astype(jnp.int32)
    disp = jnp.pad(disp, ((0, tot - n_pair), (0, 0)))
    dp = disp[:, 0]
    dbi = disp[:, 1]
    dbj = disp[:, 2]

    crd = jnp.pad(coords.reshape(n_pose * n_at, 3), ((0, 0), (0, 13)))
    bty = pose_stack_block_type.reshape(-1).astype(jnp.int32)
    off = pose_stack_block_coord_offset.reshape(-1).astype(jnp.int32)
    sep = pose_stack_min_bond_separation.reshape(-1, 16).astype(jnp.int32)
    donh = bt_tile_donH_inds.reshape(-1).astype(jnp.int32)
    accs = bt_tile_acc_inds.reshape(-1).astype(jnp.int32)
    dty = bt_tile_donor_type.reshape(-1).astype(jnp.int32)
    aty = bt_tile_acceptor_type.reshape(-1).astype(jnp.int32)
    hyb = bt_tile_acceptor_hybridization.reshape(-1).astype(jnp.int32)
    ish = bt_atom_is_hydrogen.reshape(-1).astype(jnp.int32)
    npadbt = (-n_bt) % _LN
    ndon = jnp.pad(bt_tile_n_donH.reshape(-1), (0, npadbt)).astype(jnp.int32)
    nacc = jnp.pad(bt_tile_n_acc.reshape(-1), (0, npadbt)).astype(jnp.int32)
    pathd = bt_path_distance.reshape(-1).astype(jnp.int32)
    coef = jnp.pad(pair_polynomials.reshape(ndt * nat, deg),
                   ((0, 0), (0, _CPAD - deg))).reshape(-1)
    npp16 = ndt * nat + (-ndt * nat) % _LN
    pp0 = jnp.pad(pair_params[..., 0].reshape(-1), (0, (-ndt * nat) % _LN))
    pp1 = jnp.pad(pair_params[..., 1].reshape(-1), (0, (-ndt * nat) % _LN))
    g16 = jnp.broadcast_to(global_params[0], (_LN,))

    mesh = plsc.VectorSubcoreMesh(core_axis_name="c", subcore_axis_name="s")
    f32 = jnp.float32
    i32 = jnp.int32
    run = pl.kernel(
        functools.partial(_body, n_pair, n_pose, n_blk, n_tile, n_at, wp,
                          nch_w),
        out_type=jax.ShapeDtypeStruct((_NW, n_pose), f32),
        mesh=mesh,
        compiler_params=pltpu.CompilerParams(
            needs_layout_passes=False, use_tc_tiling_on_sc=False),
        scratch_types=[
            pltpu.VMEM((wp,), i32),
            pltpu.VMEM((wp,), i32),
            pltpu.VMEM((wp,), i32),
            pltpu.VMEM((n_pose * n_blk,), i32),
            pltpu.VMEM((n_pose * n_blk,), i32),
            pltpu.VMEM((n_bt * n_tile,), i32),
            pltpu.VMEM((n_bt * n_tile,), i32),
            pltpu.VMEM((n_bt * n_tile,), i32),
            pltpu.VMEM((n_bt * n_tile,), i32),
            pltpu.VMEM((n_bt * n_tile,), i32),
            pltpu.VMEM((n_bt * n_tile,), i32),
            pltpu.VMEM((n_bt + npadbt,), i32),
            pltpu.VMEM((n_bt + npadbt,), i32),
            pltpu.VMEM((n_bt * n_tile * n_tile,), i32),
            pltpu.VMEM((ndt * nat * _CPAD,), f32),
            pltpu.VMEM((npp16,), f32),
            pltpu.VMEM((npp16,), f32),
            pltpu.VMEM((_LN,), f32),
            pltpu.VMEM((npp16,), f32),
            pltpu.VMEM((npp16,), f32),
            pltpu.VMEM((n_bt * n_tile,), f32),
            pltpu.VMEM(((_MAXD + _MAXA) * _NCH,), i32),
            pltpu.VMEM(((_MAXD + _MAXA) * _NCH,), i32),
            pltpu.VMEM((_NCH,), i32),
            pltpu.VMEM((_NCH,), i32),
            pltpu.VMEM((_NCH,), i32),
            pltpu.VMEM((_NCH,), i32),
            pltpu.VMEM((_NCH,), i32),
            pltpu.VMEM((_NCH,), i32),
            pltpu.VMEM((_MAXD + _MAXA, _NCH, 16), f32),
            pltpu.VMEM((_MAXD + _MAXA, _NCH, 16), f32),
            pltpu.VMEM((_NCH, 16), i32),
            pltpu.VMEM((_NCH, 16), i32),
            pltpu.VMEM((n_pose * _LN,), f32),
            pltpu.VMEM((n_pose,), f32),
            pltpu.SemaphoreType.DMA,
            pltpu.SemaphoreType.DMA,
        ],
    )
    partial = run(dp, dbi, dbj, bty, off, donh, accs, dty, aty, hyb, ish,
                  ndon, nacc, pathd, coef, pp0, pp1, sep, g16, crd)
    return jnp.sum(partial, axis=0)


# speedup vs baseline: 216.9176x; 1.0263x over previous
import functools

import jax
import jax.numpy as jnp
from jax import lax
from jax.experimental import pallas as pl
from jax.experimental.pallas import tpu as pltpu
from jax.experimental.pallas import tpu_sc as plsc

_NC = 2
_NS = 16
_NW = _NC * _NS
_LN = 16
_NCH = 128
_MAXD = 4
_MAXA = 4
_DEG = 11
_CPAD = 16


def _rsqrt(x):
    i = plsc.bitcast(x, jnp.int32)
    i = 0x5F3759DF - lax.shift_right_logical(i, 1)
    y = plsc.bitcast(i, jnp.float32)
    for _ in range(3):
        y = y * (1.5 - 0.5 * x * y * y)
    return y


def _body(n_pair, n_pose, n_blk, n_tile, n_at, wp, nch_w,
          dp_h, dbi_h, dbj_h, bty_h, off_h, donh_h, acc_h, dty_h, aty_h,
          hyb_h, ish_h, ndon_h, nacc_h, pathd_h, coef_h, pp0_h, pp1_h,
          sep_h, g_h, crd_h,
          out_h,
          dp_v, dbi_v, dbj_v, bty_v, off_v, donh_v, acc_v, dty_v, aty_v,
          hyb_v, ish_v, ndon_v, nacc_v, pathd_v, coef_v, pp0_v, pp1_v,
          g_v, lo_v, hi_v, fade_v,
          idxc0, idxc1, idxs0, idxs1, btib0, btib1, btjb0, btjb1,
          cbuf0, cbuf1, sbuf0, sbuf1, tot_v, orow, sem0, sem1):
    wid = lax.axis_index("s") * _NC + lax.axis_index("c")
    wbase = wid * wp

    pltpu.sync_copy(dp_h.at[pl.ds(wbase, wp)], dp_v)
    pltpu.sync_copy(dbi_h.at[pl.ds(wbase, wp)], dbi_v)
    pltpu.sync_copy(dbj_h.at[pl.ds(wbase, wp)], dbj_v)
    pltpu.sync_copy(bty_h, bty_v)
    pltpu.sync_copy(off_h, off_v)
    pltpu.sync_copy(donh_h, donh_v)
    pltpu.sync_copy(acc_h, acc_v)
    pltpu.sync_copy(dty_h, dty_v)
    pltpu.sync_copy(aty_h, aty_v)
    pltpu.sync_copy(hyb_h, hyb_v)
    pltpu.sync_copy(ish_h, ish_v)
    pltpu.sync_copy(ndon_h, ndon_v)
    pltpu.sync_copy(nacc_h, nacc_v)
    pltpu.sync_copy(pathd_h, pathd_v)
    pltpu.sync_copy(coef_h, coef_v)
    pltpu.sync_copy(pp0_h, pp0_v)
    pltpu.sync_copy(pp1_h, pp1_v)
    pltpu.sync_copy(g_h, g_v)

    iota = lax.iota(jnp.int32, _LN)
    zero16 = jnp.zeros((_LN,), jnp.float32)

    npp = lo_v.shape[0] // _LN
    for i in range(npp):
        p0 = pp0_v[pl.ds(i * _LN, _LN)]
        p1 = pp1_v[pl.ds(i * _LN, _LN)]
        lo_v[pl.ds(i * _LN, _LN)] = jnp.minimum(p0, p1)
        hi_v[pl.ds(i * _LN, _LN)] = 3.0 * (1.0 + jnp.maximum(p0, p1))

    def fade_body(i, _):
        hy = hyb_v[pl.ds(i * _LN, _LN)]
        fade_v[pl.ds(i * _LN, _LN)] = 1.0 + 0.1 * hy.astype(jnp.float32)
        return 0
    lax.fori_loop(0, fade_v.shape[0] // _LN, fade_body, 0)

    def zero_body(i, _):
        tot_v[pl.ds(i * _LN, _LN)] = zero16
        return 0
    lax.fori_loop(0, n_pose, zero_body, 0)

    nvec = _NCH // _LN

    def a_phase(cc, idxc, idxs, btib, btjb):
        cb = cc * _NCH

        @plsc.parallel_loop(0, nvec, unroll=2)
        def a_body(v):
            b = cb + v * _LN
            vb = v * _LN
            pv = dp_v[pl.ds(b, _LN)]
            biv = dbi_v[pl.ds(b, _LN)]
            bjv = dbj_v[pl.ds(b, _LN)]
            pb = pv * n_blk + biv
            pbj = pv * n_blk + bjv
            bti = plsc.load_gather(bty_v, [pb])
            btj = plsc.load_gather(bty_v, [pbj])
            offi = plsc.load_gather(off_v, [pb])
            offj = plsc.load_gather(off_v, [pbj])
            ci = pv * n_at + offi
            cj = pv * n_at + offj
            btib[pl.ds(vb, _LN)] = bti
            btjb[pl.ds(vb, _LN)] = btj
            for k in range(_MAXD):
                hk = plsc.load_gather(donh_v, [bti * n_tile + k])
                idxc[pl.ds(k * _NCH + vb, _LN)] = ci + hk
            for a in range(_MAXA):
                ak = plsc.load_gather(acc_v, [btj * n_tile + a])
                idxc[pl.ds((_MAXD + a) * _NCH + vb, _LN)] = cj + ak
            idxs[pl.ds(vb, _LN)] = lax.shift_right_logical(
                pb * n_blk + bjv, 4)

    def fire(idxc, idxs, cbuf, sbuf, sem):
        for k in range(_MAXD + _MAXA):
            pltpu.async_copy(
                crd_h.at[idxc.at[pl.ds(k * _NCH, _NCH)]], cbuf.at[k], sem)
        pltpu.async_copy(sep_h.at[idxs], sbuf, sem)

    def drain(idxc, idxs, cbuf, sbuf, sem):
        for k in range(_MAXD + _MAXA):
            pltpu.make_async_copy(
                crd_h.at[idxc.at[pl.ds(k * _NCH, _NCH)]],
                cbuf.at[k], sem).wait()
        pltpu.make_async_copy(sep_h.at[idxs], sbuf, sem).wait()

    def b_phase(cc, btib, btjb, cbuf, sbuf):
        cb = cc * _NCH

        @plsc.parallel_loop(0, nvec, unroll=1)
        def b_body(v):
            b = cb + v * _LN
            vb = v * _LN
            j16 = vb + iota
            gidx = wbase + b + iota
            valid = gidx < n_pair
            pv = dp_v[pl.ds(b, _LN)]
            biv = dbi_v[pl.ds(b, _LN)]
            bjv = dbj_v[pl.ds(b, _LN)]
            bti = btib[pl.ds(vb, _LN)]
            btj = btjb[pl.ds(vb, _LN)]
            nH = plsc.load_gather(ndon_v, [bti])
            nA = plsc.load_gather(nacc_v, [btj])
            flat = (pv * n_blk + biv) * n_blk + bjv
            sep_inter = plsc.load_gather(sbuf, [j16, flat & 15])
            intra = biv == bjv

            Hx, Hy, Hz, hidx, ishk, dtk = [], [], [], [], [], []
            for k in range(_MAXD):
                kc = jnp.full((_LN,), k, jnp.int32)
                hk = plsc.load_gather(donh_v, [bti * n_tile + k])
                hidx.append(hk)
                ishk.append(plsc.load_gather(ish_v, [bti * n_tile + hk]) > 0)
                dtk.append(plsc.load_gather(dty_v, [bti * n_tile + k]))
                Hx.append(plsc.load_gather(
                    cbuf, [kc, j16, jnp.full((_LN,), 0, jnp.int32)]))
                Hy.append(plsc.load_gather(
                    cbuf, [kc, j16, jnp.full((_LN,), 1, jnp.int32)]))
                Hz.append(plsc.load_gather(
                    cbuf, [kc, j16, jnp.full((_LN,), 2, jnp.int32)]))
            Ax, Ay, Az, aidx, atk, fade = [], [], [], [], [], []
            for a in range(_MAXA):
                ac = jnp.full((_LN,), _MAXD + a, jnp.int32)
                ak = plsc.load_gather(acc_v, [btj * n_tile + a])
                aidx.append(ak)
                atk.append(plsc.load_gather(aty_v, [btj * n_tile + a]))
                fade.append(plsc.load_gather(fade_v, [btj * n_tile + a]))
                Ax.append(plsc.load_gather(
                    cbuf, [ac, j16, jnp.full((_LN,), 0, jnp.int32)]))
                Ay.append(plsc.load_gather(
                    cbuf, [ac, j16, jnp.full((_LN,), 1, jnp.int32)]))
                Az.append(plsc.load_gather(
                    cbuf, [ac, j16, jnp.full((_LN,), 2, jnp.int32)]))

            acc = zero16
            for k in range(_MAXD):
                hv = nH > k
                for a in range(_MAXA):
                    av = nA > a
                    dx = Ax[a] - Hx[k]
                    dy = Ay[a] - Hy[k]
                    dz = Az[a] - Hz[k]
                    s = dx * dx + dy * dy + dz * dz + 1e-8
                    r = _rsqrt(s)
                    d = s * r
                    pi = dtk[k] * 6 + atk[a]
                    x = d * (1.0 / 3.0)
                    val = plsc.load_gather(coef_v, [pi * _CPAD])
                    for q in range(1, _DEG):
                        val = val * x + plsc.load_gather(
                            coef_v, [pi * _CPAD + q])
                    lo = plsc.load_gather(lo_v, [pi])
                    hi = plsc.load_gather(hi_v, [pi])
                    si = plsc.load_gather(
                        pathd_v, [(bti * n_tile + hidx[k]) * n_tile + aidx[a]])
                    sep = jnp.where(intra, si, sep_inter)
                    m = (hv & av & (d > lo) & (d < hi) & (sep >= 4)
                         & ishk[k] & valid)
                    acc = acc + jnp.where(m, val * fade[a], 0.0)
            plsc.addupdate_scatter(tot_v, [pv * _LN + iota], acc)

    a_phase(0, idxc0, idxs0, btib0, btjb0)
    fire(idxc0, idxs0, cbuf0, sbuf0, sem0)

    def pair_body(g, _):
        c0 = 2 * g
        c1 = c0 + 1
        a_phase(c1, idxc1, idxs1, btib1, btjb1)
        fire(idxc1, idxs1, cbuf1, sbuf1, sem1)
        drain(idxc0, idxs0, cbuf0, sbuf0, sem0)
        b_phase(c0, btib0, btjb0, cbuf0, sbuf0)

        @pl.when(c0 + 2 < nch_w)
        def _():
            a_phase(c0 + 2, idxc0, idxs0, btib0, btjb0)
            fire(idxc0, idxs0, cbuf0, sbuf0, sem0)
        drain(idxc1, idxs1, cbuf1, sbuf1, sem1)
        b_phase(c1, btib1, btjb1, cbuf1, sbuf1)
        return 0
    lax.fori_loop(0, nch_w // 2, pair_body, 0)

    gv = g_v[pl.ds(0, _LN)]
    for grp in range(n_pose // _LN):
        rows = grp * _LN + iota
        accg = zero16
        for j in range(_LN):
            accg = accg + plsc.load_gather(tot_v, [rows * _LN + j])
        orow[pl.ds(grp * _LN, _LN)] = accg * gv
    pltpu.sync_copy(orow, out_h.at[wid])


def kernel(coords, block_pair_dispatch_indices, pose_stack_block_coord_offset,
           pose_stack_block_type, pose_stack_min_bond_separation,
           bt_tile_n_donH, bt_tile_n_acc, bt_tile_donH_inds, bt_tile_acc_inds,
           bt_tile_donor_type, bt_tile_acceptor_type,
           bt_tile_acceptor_hybridization, bt_atom_is_hydrogen,
           bt_path_distance, pair_params, pair_polynomials, global_params):
    n_pose, n_at, _ = coords.shape
    n_blk = pose_stack_block_type.shape[1]
    n_bt, _, n_tile = bt_tile_donH_inds.shape
    n_pair = block_pair_dispatch_indices.shape[0]
    ndt, nat, deg = pair_polynomials.shape

    chunk_total = _NW * _NCH
    nch_w = -(-n_pair // chunk_total)
    nch_w += nch_w % 2
    wp = nch_w * _NCH
    tot = _NW * wp

    disp = block_pair_dispatch_indices.astype(jnp.int32)
    disp = jnp.pad(disp, ((0, tot - n_pair), (0, 0)))
    dp = disp[:, 0]
    dbi = disp[:, 1]
    dbj = disp[:, 2]

    crd = jnp.pad(coords.reshape(n_pose * n_at, 3), ((0, 0), (0, 13)))
    bty = pose_stack_block_type.reshape(-1).astype(jnp.int32)
    off = pose_stack_block_coord_offset.reshape(-1).astype(jnp.int32)
    sep = pose_stack_min_bond_separation.reshape(-1, 16).astype(jnp.int32)
    donh = bt_tile_donH_inds.reshape(-1).astype(jnp.int32)
    accs = bt_tile_acc_inds.reshape(-1).astype(jnp.int32)
    dty = bt_tile_donor_type.reshape(-1).astype(jnp.int32)
    aty = bt_tile_acceptor_type.reshape(-1).astype(jnp.int32)
    hyb = bt_tile_acceptor_hybridization.reshape(-1).astype(jnp.int32)
    ish = bt_atom_is_hydrogen.reshape(-1).astype(jnp.int32)
    npadbt = (-n_bt) % _LN
    ndon = jnp.pad(bt_tile_n_donH.reshape(-1), (0, npadbt)).astype(jnp.int32)
    nacc = jnp.pad(bt_tile_n_acc.reshape(-1), (0, npadbt)).astype(jnp.int32)
    pathd = bt_path_distance.reshape(-1).astype(jnp.int32)
    coef = jnp.pad(pair_polynomials.reshape(ndt * nat, deg),
                   ((0, 0), (0, _CPAD - deg))).reshape(-1)
    npp16 = ndt * nat + (-ndt * nat) % _LN
    pp0 = jnp.pad(pair_params[..., 0].reshape(-1), (0, (-ndt * nat) % _LN))
    pp1 = jnp.pad(pair_params[..., 1].reshape(-1), (0, (-ndt * nat) % _LN))
    g16 = jnp.broadcast_to(global_params[0], (_LN,))

    mesh = plsc.VectorSubcoreMesh(core_axis_name="c", subcore_axis_name="s")
    f32 = jnp.float32
    i32 = jnp.int32
    run = pl.kernel(
        functools.partial(_body, n_pair, n_pose, n_blk, n_tile, n_at, wp,
                          nch_w),
        out_type=jax.ShapeDtypeStruct((_NW, n_pose), f32),
        mesh=mesh,
        compiler_params=pltpu.CompilerParams(
            needs_layout_passes=False, use_tc_tiling_on_sc=False),
        scratch_types=[
            pltpu.VMEM((wp,), i32),
            pltpu.VMEM((wp,), i32),
            pltpu.VMEM((wp,), i32),
            pltpu.VMEM((n_pose * n_blk,), i32),
            pltpu.VMEM((n_pose * n_blk,), i32),
            pltpu.VMEM((n_bt * n_tile,), i32),
            pltpu.VMEM((n_bt * n_tile,), i32),
            pltpu.VMEM((n_bt * n_tile,), i32),
            pltpu.VMEM((n_bt * n_tile,), i32),
            pltpu.VMEM((n_bt * n_tile,), i32),
            pltpu.VMEM((n_bt * n_tile,), i32),
            pltpu.VMEM((n_bt + npadbt,), i32),
            pltpu.VMEM((n_bt + npadbt,), i32),
            pltpu.VMEM((n_bt * n_tile * n_tile,), i32),
            pltpu.VMEM((ndt * nat * _CPAD,), f32),
            pltpu.VMEM((npp16,), f32),
            pltpu.VMEM((npp16,), f32),
            pltpu.VMEM((_LN,), f32),
            pltpu.VMEM((npp16,), f32),
            pltpu.VMEM((npp16,), f32),
            pltpu.VMEM((n_bt * n_tile,), f32),
            pltpu.VMEM(((_MAXD + _MAXA) * _NCH,), i32),
            pltpu.VMEM(((_MAXD + _MAXA) * _NCH,), i32),
            pltpu.VMEM((_NCH,), i32),
            pltpu.VMEM((_NCH,), i32),
            pltpu.VMEM((_NCH,), i32),
            pltpu.VMEM((_NCH,), i32),
            pltpu.VMEM((_NCH,), i32),
            pltpu.VMEM((_NCH,), i32),
            pltpu.VMEM((_MAXD + _MAXA, _NCH, 16), f32),
            pltpu.VMEM((_MAXD + _MAXA, _NCH, 16), f32),
            pltpu.VMEM((_NCH, 16), i32),
            pltpu.VMEM((_NCH, 16), i32),
            pltpu.VMEM((n_pose * _LN,), f32),
            pltpu.VMEM((n_pose,), f32),
            pltpu.SemaphoreType.DMA,
            pltpu.SemaphoreType.DMA,
        ],
    )
    partial = run(dp, dbi, dbj, bty, off, donh, accs, dty, aty, hyb, ish,
                  ndon, nacc, pathd, coef, pp0, pp1, sep, g16, crd)
    return jnp.sum(partial, axis=0)
